# Initial kernel scaffold; baseline (speedup 1.0000x reference)
import jax
import jax.numpy as jnp
from jax.experimental import pallas as pl


def kernel(x, Wt, Wu, M):
    raise NotImplementedError("write your pallas kernel here")


# speedup vs baseline: 1.8956x; 1.8956x over previous
import functools

import jax
import jax.numpy as jnp
from jax import lax
from jax.experimental import pallas as pl
from jax.experimental.pallas import tpu as pltpu
from jax.experimental.pallas import tpu_sc as plsc

D = 32
L = 16
NC, NS = 2, 16
NW = NC * NS

GW = 128
KR = 8
K = KR * GW


def _make_kernel(ntot):
    per_w = ntot // NW
    steps = per_w // K
    xrows_per_w = per_w // GW

    mesh = plsc.VectorSubcoreMesh(core_axis_name="c", subcore_axis_name="s")

    @functools.partial(
        pl.kernel,
        mesh=mesh,
        compiler_params=pltpu.CompilerParams(use_tc_tiling_on_sc=False),
        out_type=jax.ShapeDtypeStruct((ntot, D), jnp.float32),
        scratch_types=[
            pltpu.VMEM((KR, GW), jnp.int32),
            pltpu.VMEM((K, D), jnp.float32),
            pltpu.VMEM((K, D), jnp.float32),
            pltpu.VMEM((K, D), jnp.float32),
            pltpu.SemaphoreType.DMA,
            pltpu.SemaphoreType.DMA,
            pltpu.SemaphoreType.DMA,
        ],
    )
    def emb(x_hbm, wt_hbm, wu_hbm, m_hbm, out_hbm, idx_v, a_v, m_v, u_v,
            s0, s1, s2):
        wid = lax.axis_index("s") * NC + lax.axis_index("c")
        row_base = wid * per_w
        xrow_base = wid * xrows_per_w

        @pl.loop(0, steps)
        def _(g):
            pltpu.sync_copy(x_hbm.at[pl.ds(xrow_base + g * KR, KR)], idx_v)

            copies = []
            for j in range(KR):
                dst = pl.ds(j * GW, GW)
                copies.append(
                    pltpu.async_copy(wt_hbm.at[idx_v.at[j]], a_v.at[dst], s0))
                copies.append(
                    pltpu.async_copy(m_hbm.at[idx_v.at[j]], m_v.at[dst], s1))
                copies.append(
                    pltpu.async_copy(wu_hbm.at[idx_v.at[j]], u_v.at[dst], s2))
            for c in copies:
                c.wait()

            @pl.loop(0, K, step=4)
            def _(r):
                for dr in range(4):
                    for c0 in range(0, D, L):
                        sl = pl.ds(c0, L)
                        a_v[r + dr, sl] = (a_v[r + dr, sl] * m_v[r + dr, sl]
                                           + u_v[r + dr, sl])

            pltpu.sync_copy(a_v, out_hbm.at[pl.ds(row_base + g * K, K)])

    return emb


def kernel(x, Wt, Wu, M):
    b, h = x.shape
    ntot = b * h
    x2d = x.reshape(ntot // GW, GW)
    out = _make_kernel(ntot)(x2d, Wt, Wu, M)
    return out.reshape(b, h, D)


# speedup vs baseline: 2.7521x; 1.4518x over previous
import functools

import jax
import jax.numpy as jnp
from jax import lax
from jax.experimental import pallas as pl
from jax.experimental.pallas import tpu as pltpu
from jax.experimental.pallas import tpu_sc as plsc

D = 32
DP = 128
NC, NS = 2, 16
NW = NC * NS

GW = 128


def _combine_tables(wtT, mT, wuT):
    vocab = wtT.shape[1]
    blk = 2048

    def body(a_ref, m_ref, u_ref, o_ref):
        v = a_ref[...] * m_ref[...] + u_ref[...]
        o_ref[pl.ds(0, blk), pl.ds(0, D)] = jnp.transpose(v, (1, 0))

    in_spec = pl.BlockSpec((D, blk), lambda i: (0, i))
    return pl.pallas_call(
        body,
        grid=(pl.cdiv(vocab, blk),),
        in_specs=[in_spec, in_spec, in_spec],
        out_specs=pl.BlockSpec((blk, DP), lambda i: (i, 0)),
        out_shape=jax.ShapeDtypeStruct((vocab, DP), jnp.float32),
    )(wtT, mT, wuT)


def _make_gather(ntot):
    per_w = ntot // NW
    K = 256
    steps = per_w // K
    KR = K // GW
    xr_per_w = per_w // GW

    mesh = plsc.VectorSubcoreMesh(core_axis_name="c", subcore_axis_name="s")

    @functools.partial(
        pl.kernel,
        mesh=mesh,
        compiler_params=pltpu.CompilerParams(use_tc_tiling_on_sc=False),
        out_type=jax.ShapeDtypeStruct((ntot, D), jnp.float32),
        scratch_types=[
            pltpu.VMEM((KR, GW), jnp.int32),
            pltpu.VMEM((KR, GW), jnp.int32),
            pltpu.VMEM((K, DP), jnp.float32),
            pltpu.VMEM((K, DP), jnp.float32),
            pltpu.SemaphoreType.DMA,
            pltpu.SemaphoreType.DMA,
            pltpu.SemaphoreType.DMA,
            pltpu.SemaphoreType.DMA,
        ],
    )
    def gk(x_hbm, v_hbm, out_hbm, i0, i1, b0, b1, g0, g1, w0, w1):
        wid = lax.axis_index("s") * NC + lax.axis_index("c")
        rbase = wid * per_w
        xbase = wid * xr_per_w
        slots = ((i0, b0, g0, w0), (i1, b1, g1, w1))

        @pl.loop(0, steps // 2)
        def _(p):
            for s, (ix, buf, gs, ws) in enumerate(slots):
                g = p * 2 + s

                @pl.when(p > 0)
                def _():
                    pltpu.make_async_copy(
                        buf.at[pl.ds(0, K), pl.ds(0, D)],
                        out_hbm.at[pl.ds(rbase, K)], ws).wait()

                pltpu.sync_copy(x_hbm.at[pl.ds(xbase + g * KR, KR)], ix)
                for j in range(KR):
                    pltpu.make_async_copy(
                        v_hbm.at[ix.at[j]],
                        buf.at[pl.ds(j * GW, GW)], gs).start()
                for j in range(KR):
                    pltpu.make_async_copy(
                        v_hbm.at[ix.at[j]],
                        buf.at[pl.ds(j * GW, GW)], gs).wait()
                pltpu.make_async_copy(
                    buf.at[pl.ds(0, K), pl.ds(0, D)],
                    out_hbm.at[pl.ds(rbase + g * K, K)], ws).start()

        for s, (ix, buf, gs, ws) in enumerate(slots):
            pltpu.make_async_copy(
                buf.at[pl.ds(0, K), pl.ds(0, D)],
                out_hbm.at[pl.ds(rbase, K)], ws).wait()

    return gk


def kernel(x, Wt, Wu, M):
    b, h = x.shape
    ntot = b * h
    v = _combine_tables(Wt.T, M.T, Wu.T)
    x2d = x.reshape(ntot // GW, GW)
    out = _make_gather(ntot)(x2d, v)
    return out.reshape(b, h, D)


# speedup vs baseline: 3.5922x; 1.3052x over previous
import functools

import jax
import jax.numpy as jnp
from jax import lax
from jax.experimental import pallas as pl
from jax.experimental.pallas import tpu as pltpu
from jax.experimental.pallas import tpu_sc as plsc

D = 32
DP = 128
NC, NS = 2, 16
NW = NC * NS

GW = 128


def _combine_tables(wtT, mT, wuT):
    vocab = wtT.shape[1]
    blk = 2048

    def body(a_ref, m_ref, u_ref, o_ref):
        v = a_ref[...] * m_ref[...] + u_ref[...]
        o_ref[pl.ds(0, blk), pl.ds(0, D)] = jnp.transpose(v, (1, 0))

    in_spec = pl.BlockSpec((D, blk), lambda i: (0, i))
    return pl.pallas_call(
        body,
        grid=(pl.cdiv(vocab, blk),),
        in_specs=[in_spec, in_spec, in_spec],
        out_specs=pl.BlockSpec((blk, DP), lambda i: (i, 0)),
        out_shape=jax.ShapeDtypeStruct((vocab, DP), jnp.float32),
    )(wtT, mT, wuT)


def _make_gather(ntot):
    per_w = ntot // NW
    K = 1280
    steps = per_w // K
    KR = K // GW
    xr_per_w = per_w // GW

    mesh = plsc.VectorSubcoreMesh(core_axis_name="c", subcore_axis_name="s")

    @functools.partial(
        pl.kernel,
        mesh=mesh,
        compiler_params=pltpu.CompilerParams(use_tc_tiling_on_sc=False),
        out_type=jax.ShapeDtypeStruct((ntot, D), jnp.float32),
        scratch_types=[
            pltpu.VMEM((KR, GW), jnp.int32),
            pltpu.VMEM((KR, GW), jnp.int32),
            pltpu.VMEM((K, D), jnp.float32),
            pltpu.VMEM((K, D), jnp.float32),
            pltpu.SemaphoreType.DMA,
            pltpu.SemaphoreType.DMA,
            pltpu.SemaphoreType.DMA,
            pltpu.SemaphoreType.DMA,
        ],
    )
    def gk(x_hbm, v_hbm, out_hbm, i0, i1, b0, b1, g0, g1, w0, w1):
        wid = lax.axis_index("s") * NC + lax.axis_index("c")
        rbase = wid * per_w
        xbase = wid * xr_per_w
        slots = ((i0, b0, g0, w0), (i1, b1, g1, w1))

        @pl.loop(0, steps // 2)
        def _(p):
            for s, (ix, buf, gs, ws) in enumerate(slots):
                g = p * 2 + s

                @pl.when(p > 0)
                def _():
                    pltpu.make_async_copy(
                        buf, out_hbm.at[pl.ds(rbase, K)], ws).wait()

                pltpu.sync_copy(x_hbm.at[pl.ds(xbase + g * KR, KR)], ix)
                for j in range(KR):
                    pltpu.make_async_copy(
                        v_hbm.at[ix.at[j]],
                        buf.at[pl.ds(j * GW, GW)], gs).start()
                for j in range(KR):
                    pltpu.make_async_copy(
                        v_hbm.at[ix.at[j]],
                        buf.at[pl.ds(j * GW, GW)], gs).wait()
                pltpu.make_async_copy(
                    buf, out_hbm.at[pl.ds(rbase + g * K, K)], ws).start()

        for s, (ix, buf, gs, ws) in enumerate(slots):
            pltpu.make_async_copy(
                buf, out_hbm.at[pl.ds(rbase, K)], ws).wait()

    return gk


def kernel(x, Wt, Wu, M):
    b, h = x.shape
    ntot = b * h
    v = _combine_tables(Wt.T, M.T, Wu.T)
    v4 = v.reshape(4 * v.shape[0], D)
    x4 = x.reshape(ntot // GW, GW) * 4
    out = _make_gather(ntot)(x4, v4)
    return out.reshape(b, h, D)
